# baseline (speedup 1.0000x reference)
import functools

import jax
import jax.numpy as jnp
from jax import lax
from jax.experimental import pallas as pl
from jax.experimental.pallas import tpu as pltpu
from jax.experimental.pallas import tpu_sc as plsc

F = 128
G = 64
NC, NS = 2, 16
NW = NC * NS
CH = 128
VR = F // 16
BM = 512

_MESH = dict(core_axis_name="c", subcore_axis_name="s", num_cores=NC,
             num_subcores=NS)
_SC_PARAMS = pltpu.CompilerParams(needs_layout_passes=False)


def _deg_body(n_pad, ng, col_ref, w_ref, out_ref, colv, wv, zv, deg_sh):
    c = lax.axis_index("c")
    s = lax.axis_index("s")
    wid = s * NC + c
    rpt = n_pad // NS
    pltpu.sync_copy(col_ref.at[wid], colv)
    pltpu.sync_copy(w_ref.at[wid], wv)

    def zero(i, carry):
        zv[pl.ds(i * 16, 16)] = jnp.zeros((16,), jnp.float32)
        return carry
    lax.fori_loop(0, rpt // 16, zero, None)
    pltpu.sync_copy(zv, deg_sh.at[pl.ds(s * rpt, rpt)])
    plsc.subcore_barrier()

    def group(g, carry):
        def chunk(j, c1):
            pltpu.sync_copy(wv.at[g, j], deg_sh.at[colv.at[g, j]], add=True)
            return c1
        return lax.fori_loop(0, Q, chunk, carry)
    lax.fori_loop(0, ng, group, None)
    plsc.subcore_barrier()
    pltpu.sync_copy(deg_sh.at[pl.ds(s * rpt, rpt)],
                    out_ref.at[c, pl.ds(s * rpt, rpt)])


def _make_deg(n_pad, ng):
    return pl.kernel(
        functools.partial(_deg_body, n_pad, ng),
        out_type=jax.ShapeDtypeStruct((NC, n_pad), jnp.float32),
        mesh=plsc.VectorSubcoreMesh(**_MESH),
        scratch_types=[
            pltpu.VMEM((ng, Q, CH), jnp.int32),
            pltpu.VMEM((ng, Q, CH), jnp.float32),
            pltpu.VMEM((n_pad // NS,), jnp.float32),
            pltpu.VMEM_SHARED((n_pad,), jnp.float32),
        ],
        compiler_params=_SC_PARAMS,
    )


Q = 8
ZR = 32


def _scale_chunk(gbuf, wnv, j):
    def egrp(g2, c2):
        w16 = wnv[j, pl.ds(g2 * 16, 16)]
        for i in range(16):
            wsc = w16[i]
            row = g2 * 16 + i
            for k in range(VR):
                sl = pl.ds(k * 16, 16)
                gbuf[row, sl] = gbuf[row, sl] * wsc
        return c2
    lax.fori_loop(0, CH // 16, egrp, None)


def _layer_body(n_pad, ng, compute_wn, *refs):
    if compute_wn:
        (src_ref, row_ref, col_ref, w_ref, dinv_ref, agg_out, wn_out,
         rowv, colv, wnv, dinvv, gbuf0, gbuf1, sem0, sem1, ss0, ss1,
         agg_sh) = refs
    else:
        (src_ref, row_ref, col_ref, w_ref, agg_out,
         rowv, colv, wnv, gbuf0, gbuf1, sem0, sem1, ss0, ss1, agg_sh) = refs
    c = lax.axis_index("c")
    s = lax.axis_index("s")
    wid = s * NC + c
    rpt = n_pad // NS
    bufs = (gbuf0, gbuf1)
    sems = (sem0, sem1)
    ssems = (ss0, ss1)

    def zrow(r, carry):
        for k in range(VR):
            gbuf0[r, pl.ds(k * 16, 16)] = jnp.zeros((16,), jnp.float32)
        return carry
    lax.fori_loop(0, CH, zrow, None)
    for t in range(rpt // CH):
        pltpu.sync_copy(gbuf0, agg_sh.at[pl.ds(s * rpt + t * CH, CH)])
    if compute_wn:
        pltpu.sync_copy(dinv_ref, dinvv)
    plsc.subcore_barrier()

    def group(g, carry):
        pltpu.sync_copy(row_ref.at[wid, g], rowv)
        pltpu.sync_copy(col_ref.at[wid, g], colv)
        pltpu.sync_copy(w_ref.at[wid, g], wnv)

        if compute_wn:
            def wnchunk(j, c1):
                def grp(k, c2):
                    sl = pl.ds(k * 16, 16)
                    dr = plsc.load_gather(dinvv, [rowv[j, sl]])
                    dc = plsc.load_gather(dinvv, [colv[j, sl]])
                    wnv[j, sl] = wnv[j, sl] * dr * dc
                    return c2
                return lax.fori_loop(0, CH // 16, grp, c1)
            lax.fori_loop(0, Q, wnchunk, None)
            pltpu.sync_copy(wnv, wn_out.at[wid, g])

        pltpu.async_copy(src_ref.at[rowv.at[0]], bufs[0], sems[0])
        for j in range(Q):
            p = j & 1
            pltpu.make_async_copy(src_ref.at[rowv.at[j]], bufs[p],
                                  sems[p]).wait()
            _scale_chunk(bufs[p], wnv, j)
            pltpu.async_copy(bufs[p], agg_sh.at[colv.at[j]], ssems[p],
                             add=True)
            if j + 1 < Q:
                pn = (j + 1) & 1
                if j >= 1:
                    pltpu.make_async_copy(bufs[pn],
                                          agg_sh.at[colv.at[j - 1]],
                                          ssems[pn]).wait()
                pltpu.async_copy(src_ref.at[rowv.at[j + 1]], bufs[pn],
                                 sems[pn])
        for j in range(max(0, Q - 2), Q):
            p = j & 1
            pltpu.make_async_copy(bufs[p], agg_sh.at[colv.at[j]],
                                  ssems[p]).wait()
        return carry
    lax.fori_loop(0, ng, group, None)

    plsc.subcore_barrier()
    pltpu.sync_copy(agg_sh.at[pl.ds(s * rpt, rpt)], agg_out.at[c, s])


def _make_layer(n_pad, ng, compute_wn):
    out_type = [jax.ShapeDtypeStruct((NC, NS, n_pad // NS, F), jnp.float32)]
    if compute_wn:
        out_type.append(jax.ShapeDtypeStruct((NW, ng, Q, CH), jnp.float32))
    scratch = [
        pltpu.VMEM((Q, CH), jnp.int32),
        pltpu.VMEM((Q, CH), jnp.int32),
        pltpu.VMEM((Q, CH), jnp.float32),
    ]
    if compute_wn:
        scratch.append(pltpu.VMEM((n_pad,), jnp.float32))
    scratch += [
        pltpu.VMEM((CH, F), jnp.float32),
        pltpu.VMEM((CH, F), jnp.float32),
        pltpu.SemaphoreType.DMA,
        pltpu.SemaphoreType.DMA,
        pltpu.SemaphoreType.DMA,
        pltpu.SemaphoreType.DMA,
        pltpu.VMEM_SHARED((n_pad, F), jnp.float32),
    ]
    return pl.kernel(
        functools.partial(_layer_body, n_pad, ng, compute_wn),
        out_type=out_type,
        mesh=plsc.VectorSubcoreMesh(**_MESH),
        scratch_types=scratch,
        compiler_params=_SC_PARAMS,
    )


def _prep_body(x_ref, iw0_ref, rw0_ref, degp_ref, out0_ref, root0_ref,
               dinv_ref):
    xb = x_ref[...]
    out0_ref[...] = jnp.dot(xb, iw0_ref[...],
                            preferred_element_type=jnp.float32)
    root0_ref[...] = jnp.dot(xb, rw0_ref[...],
                             preferred_element_type=jnp.float32)
    deg = jnp.sum(degp_ref[...], axis=0)
    dinv = jnp.where(deg > 0, lax.rsqrt(jnp.maximum(deg, 1e-12)),
                     jnp.float32(0))
    dinv_ref[...] = dinv[None, :]


def _make_prep(n_pad):
    return pl.pallas_call(
        _prep_body,
        grid=(n_pad // BM,),
        in_specs=[
            pl.BlockSpec((BM, F), lambda i: (i, 0)),
            pl.BlockSpec((F, F), lambda i: (0, 0)),
            pl.BlockSpec((F, F), lambda i: (0, 0)),
            pl.BlockSpec((NC, BM), lambda i: (0, i)),
        ],
        out_specs=[
            pl.BlockSpec((BM, F), lambda i: (i, 0)),
            pl.BlockSpec((BM, F), lambda i: (i, 0)),
            pl.BlockSpec((1, BM), lambda i: (0, i)),
        ],
        out_shape=[
            jax.ShapeDtypeStruct((n_pad, F), jnp.float32),
            jax.ShapeDtypeStruct((n_pad, F), jnp.float32),
            jax.ShapeDtypeStruct((1, n_pad), jnp.float32),
        ],
    )


def _mid_body(aggp_ref, root0_ref, b0_ref, iw1_ref, rw1_ref, out1_ref,
              root1_ref):
    h = jnp.maximum(aggp_ref[0] + aggp_ref[1] + root0_ref[...] + b0_ref[...],
                    0.0)
    out1_ref[...] = jnp.dot(h, iw1_ref[...],
                            preferred_element_type=jnp.float32)
    root1_ref[...] = jnp.dot(h, rw1_ref[...],
                             preferred_element_type=jnp.float32)


def _make_mid(n_pad):
    return pl.pallas_call(
        _mid_body,
        grid=(n_pad // BM,),
        in_specs=[
            pl.BlockSpec((NC, BM, F), lambda i: (0, i, 0)),
            pl.BlockSpec((BM, F), lambda i: (i, 0)),
            pl.BlockSpec((1, F), lambda i: (0, 0)),
            pl.BlockSpec((F, F), lambda i: (0, 0)),
            pl.BlockSpec((F, F), lambda i: (0, 0)),
        ],
        out_specs=[
            pl.BlockSpec((BM, F), lambda i: (i, 0)),
            pl.BlockSpec((BM, F), lambda i: (i, 0)),
        ],
        out_shape=[
            jax.ShapeDtypeStruct((n_pad, F), jnp.float32),
            jax.ShapeDtypeStruct((n_pad, F), jnp.float32),
        ],
    )


def _final_body(aggp_ref, root1_ref, b1r_ref, batch_ref, w1_ref, b1m_ref,
                w2_ref, b2_ref, out_ref, sums_sc, cnts_sc):
    i = pl.program_id(0)
    nx = jnp.maximum(aggp_ref[0] + aggp_ref[1] + root1_ref[...]
                     + b1r_ref[...], 0.0)
    bb = batch_ref[...]
    oh = (lax.broadcasted_iota(jnp.int32, (G, BM), 0) == bb
          ).astype(jnp.float32)
    psum = jnp.dot(oh, nx, preferred_element_type=jnp.float32)
    pcnt = jnp.sum(oh, axis=1, keepdims=True)

    @pl.when(i == 0)
    def _():
        sums_sc[...] = jnp.zeros_like(sums_sc)
        cnts_sc[...] = jnp.zeros_like(cnts_sc)

    sums_sc[...] += psum
    cnts_sc[...] += jnp.broadcast_to(pcnt, cnts_sc.shape)

    @pl.when(i == pl.num_programs(0) - 1)
    def _():
        gx = sums_sc[...] / jnp.maximum(cnts_sc[...], 1.0)
        hm = jnp.maximum(jnp.dot(gx, w1_ref[...],
                                 preferred_element_type=jnp.float32)
                         + b1m_ref[...], 0.0)
        out_ref[...] = jnp.dot(hm, w2_ref[...],
                               preferred_element_type=jnp.float32) + b2_ref[...]


def _make_final(n_pad, n_hid2, n_cls):
    return pl.pallas_call(
        _final_body,
        grid=(n_pad // BM,),
        in_specs=[
            pl.BlockSpec((NC, BM, F), lambda i: (0, i, 0)),
            pl.BlockSpec((BM, F), lambda i: (i, 0)),
            pl.BlockSpec((1, F), lambda i: (0, 0)),
            pl.BlockSpec((1, BM), lambda i: (0, i)),
            pl.BlockSpec((F, n_hid2), lambda i: (0, 0)),
            pl.BlockSpec((1, n_hid2), lambda i: (0, 0)),
            pl.BlockSpec((n_hid2, n_cls), lambda i: (0, 0)),
            pl.BlockSpec((1, n_cls), lambda i: (0, 0)),
        ],
        out_specs=pl.BlockSpec((G, n_cls), lambda i: (0, 0)),
        out_shape=jax.ShapeDtypeStruct((G, n_cls), jnp.float32),
        scratch_shapes=[
            pltpu.VMEM((G, F), jnp.float32),
            pltpu.VMEM((G, F), jnp.float32),
        ],
    )


def kernel(x, edge_index, edge_attr, batch, init_w0, root_w0, bias0,
           init_w1, root_w1, bias1, W1, b1, W2, b2):
    n = x.shape[0]
    e = edge_index.shape[1]
    n_hid2 = W1.shape[1]
    n_cls = W2.shape[1]

    n_pad = -(-n // (NS * CH)) * (NS * CH)
    ng = -(-e // (NW * Q * CH))
    e_pad = ng * Q * CH * NW

    w = edge_attr.reshape(-1)
    row_p = jnp.pad(edge_index[0], (0, e_pad - e)).reshape(NW, ng, Q, CH)
    col_p = jnp.pad(edge_index[1], (0, e_pad - e)).reshape(NW, ng, Q, CH)
    w_p = jnp.pad(w, (0, e_pad - e)).reshape(NW, ng, Q, CH)
    x_p = jnp.pad(x, ((0, n_pad - n), (0, 0)))
    batch_p = jnp.pad(batch, (0, n_pad - n),
                      constant_values=G).reshape(1, n_pad)

    degp = _make_deg(n_pad, ng)(col_p, w_p)
    out0, root0, dinv2 = _make_prep(n_pad)(x_p, init_w0, root_w0, degp)
    agg0, wn = _make_layer(n_pad, ng, True)(
        out0, row_p, col_p, w_p, dinv2.reshape(n_pad))
    agg0 = agg0.reshape(NC, n_pad, F)
    out1, root1 = _make_mid(n_pad)(agg0, root0, bias0.reshape(1, F),
                                   init_w1, root_w1)
    agg1, = _make_layer(n_pad, ng, False)(out1, row_p, col_p, wn)
    agg1 = agg1.reshape(NC, n_pad, F)
    logits = _make_final(n_pad, n_hid2, n_cls)(
        agg1, root1, bias1.reshape(1, F), batch_p, W1, b1.reshape(1, n_hid2),
        W2, b2.reshape(1, n_cls))
    return logits


# speedup vs baseline: 1.0672x; 1.0672x over previous
import functools

import jax
import jax.numpy as jnp
from jax import lax
from jax.experimental import pallas as pl
from jax.experimental.pallas import tpu as pltpu
from jax.experimental.pallas import tpu_sc as plsc

F = 128
G = 64
NC, NS = 2, 16
NW = NC * NS
CH = 128
VR = F // 16
BM = 512

_MESH = dict(core_axis_name="c", subcore_axis_name="s", num_cores=NC,
             num_subcores=NS)
_SC_PARAMS = pltpu.CompilerParams(needs_layout_passes=False)


def _deg_body(n_pad, ng, col_ref, w_ref, out_ref, colv, wv, zv, deg_sh):
    c = lax.axis_index("c")
    s = lax.axis_index("s")
    wid = s * NC + c
    rpt = n_pad // NS
    pltpu.sync_copy(col_ref.at[wid], colv)
    pltpu.sync_copy(w_ref.at[wid], wv)

    def zero(i, carry):
        zv[pl.ds(i * 16, 16)] = jnp.zeros((16,), jnp.float32)
        return carry
    lax.fori_loop(0, rpt // 16, zero, None)
    pltpu.sync_copy(zv, deg_sh.at[pl.ds(s * rpt, rpt)])
    plsc.subcore_barrier()

    def group(g, carry):
        def chunk(j, c1):
            pltpu.sync_copy(wv.at[g, j], deg_sh.at[colv.at[g, j]], add=True)
            return c1
        return lax.fori_loop(0, Q, chunk, carry)
    lax.fori_loop(0, ng, group, None)
    plsc.subcore_barrier()
    pltpu.sync_copy(deg_sh.at[pl.ds(s * rpt, rpt)],
                    out_ref.at[c, pl.ds(s * rpt, rpt)])


def _make_deg(n_pad, ng):
    return pl.kernel(
        functools.partial(_deg_body, n_pad, ng),
        out_type=jax.ShapeDtypeStruct((NC, n_pad), jnp.float32),
        mesh=plsc.VectorSubcoreMesh(**_MESH),
        scratch_types=[
            pltpu.VMEM((ng, Q, CH), jnp.int32),
            pltpu.VMEM((ng, Q, CH), jnp.float32),
            pltpu.VMEM((n_pad // NS,), jnp.float32),
            pltpu.VMEM_SHARED((n_pad,), jnp.float32),
        ],
        compiler_params=_SC_PARAMS,
    )


Q = 8
ZR = 32


def _scale_chunk(gbuf, wnv, j):
    def egrp(g2, c2):
        w16 = wnv[j, pl.ds(g2 * 16, 16)]
        for i in range(16):
            wsc = w16[i]
            row = g2 * 16 + i
            for k in range(VR):
                sl = pl.ds(k * 16, 16)
                gbuf[row, sl] = gbuf[row, sl] * wsc
        return c2
    lax.fori_loop(0, CH // 16, egrp, None)


def _layer_body(n_pad, ng, compute_wn, *refs):
    if compute_wn:
        (src_ref, row_ref, col_ref, w_ref, dinv_ref, agg_out, wn_out,
         rowv, colv, wnv, dinvv, gbuf0, gbuf1, sem0, sem1, agg_sh) = refs
    else:
        (src_ref, row_ref, col_ref, w_ref, agg_out,
         rowv, colv, wnv, gbuf0, gbuf1, sem0, sem1, agg_sh) = refs
    c = lax.axis_index("c")
    s = lax.axis_index("s")
    wid = s * NC + c
    rpt = n_pad // NS
    bufs = (gbuf0, gbuf1)
    sems = (sem0, sem1)

    def zrow(r, carry):
        for k in range(VR):
            gbuf0[r, pl.ds(k * 16, 16)] = jnp.zeros((16,), jnp.float32)
        return carry
    lax.fori_loop(0, CH, zrow, None)
    for t in range(rpt // CH):
        pltpu.sync_copy(gbuf0, agg_sh.at[pl.ds(s * rpt + t * CH, CH)])
    if compute_wn:
        pltpu.sync_copy(dinv_ref, dinvv)
    plsc.subcore_barrier()

    def group(g, carry):
        pltpu.sync_copy(row_ref.at[wid, g], rowv)
        pltpu.sync_copy(col_ref.at[wid, g], colv)
        pltpu.sync_copy(w_ref.at[wid, g], wnv)

        if compute_wn:
            def wnchunk(j, c1):
                def grp(k, c2):
                    sl = pl.ds(k * 16, 16)
                    dr = plsc.load_gather(dinvv, [rowv[j, sl]])
                    dc = plsc.load_gather(dinvv, [colv[j, sl]])
                    wnv[j, sl] = wnv[j, sl] * dr * dc
                    return c2
                return lax.fori_loop(0, CH // 16, grp, c1)
            lax.fori_loop(0, Q, wnchunk, None)
            pltpu.sync_copy(wnv, wn_out.at[wid, g])

        pltpu.async_copy(src_ref.at[rowv.at[0]], bufs[0], sems[0])
        for j in range(Q):
            p = j & 1
            pltpu.make_async_copy(src_ref.at[rowv.at[j]], bufs[p],
                                  sems[p]).wait()
            if j + 1 < Q:
                pn = (j + 1) & 1
                pltpu.async_copy(src_ref.at[rowv.at[j + 1]], bufs[pn],
                                 sems[pn])
            _scale_chunk(bufs[p], wnv, j)
            pltpu.sync_copy(bufs[p], agg_sh.at[colv.at[j]], add=True)
        return carry
    lax.fori_loop(0, ng, group, None)

    plsc.subcore_barrier()
    pltpu.sync_copy(agg_sh.at[pl.ds(s * rpt, rpt)], agg_out.at[c, s])


def _make_layer(n_pad, ng, compute_wn):
    out_type = [jax.ShapeDtypeStruct((NC, NS, n_pad // NS, F), jnp.float32)]
    if compute_wn:
        out_type.append(jax.ShapeDtypeStruct((NW, ng, Q, CH), jnp.float32))
    scratch = [
        pltpu.VMEM((Q, CH), jnp.int32),
        pltpu.VMEM((Q, CH), jnp.int32),
        pltpu.VMEM((Q, CH), jnp.float32),
    ]
    if compute_wn:
        scratch.append(pltpu.VMEM((n_pad,), jnp.float32))
    scratch += [
        pltpu.VMEM((CH, F), jnp.float32),
        pltpu.VMEM((CH, F), jnp.float32),
        pltpu.SemaphoreType.DMA,
        pltpu.SemaphoreType.DMA,
        pltpu.VMEM_SHARED((n_pad, F), jnp.float32),
    ]
    return pl.kernel(
        functools.partial(_layer_body, n_pad, ng, compute_wn),
        out_type=out_type,
        mesh=plsc.VectorSubcoreMesh(**_MESH),
        scratch_types=scratch,
        compiler_params=_SC_PARAMS,
    )


def _prep_body(x_ref, iw0_ref, rw0_ref, degp_ref, out0_ref, root0_ref,
               dinv_ref):
    xb = x_ref[...]
    out0_ref[...] = jnp.dot(xb, iw0_ref[...],
                            preferred_element_type=jnp.float32)
    root0_ref[...] = jnp.dot(xb, rw0_ref[...],
                             preferred_element_type=jnp.float32)
    deg = jnp.sum(degp_ref[...], axis=0)
    dinv = jnp.where(deg > 0, lax.rsqrt(jnp.maximum(deg, 1e-12)),
                     jnp.float32(0))
    dinv_ref[...] = dinv[None, :]


def _make_prep(n_pad):
    return pl.pallas_call(
        _prep_body,
        grid=(n_pad // BM,),
        in_specs=[
            pl.BlockSpec((BM, F), lambda i: (i, 0)),
            pl.BlockSpec((F, F), lambda i: (0, 0)),
            pl.BlockSpec((F, F), lambda i: (0, 0)),
            pl.BlockSpec((NC, BM), lambda i: (0, i)),
        ],
        out_specs=[
            pl.BlockSpec((BM, F), lambda i: (i, 0)),
            pl.BlockSpec((BM, F), lambda i: (i, 0)),
            pl.BlockSpec((1, BM), lambda i: (0, i)),
        ],
        out_shape=[
            jax.ShapeDtypeStruct((n_pad, F), jnp.float32),
            jax.ShapeDtypeStruct((n_pad, F), jnp.float32),
            jax.ShapeDtypeStruct((1, n_pad), jnp.float32),
        ],
    )


def _mid_body(aggp_ref, root0_ref, b0_ref, iw1_ref, rw1_ref, out1_ref,
              root1_ref):
    h = jnp.maximum(aggp_ref[0] + aggp_ref[1] + root0_ref[...] + b0_ref[...],
                    0.0)
    out1_ref[...] = jnp.dot(h, iw1_ref[...],
                            preferred_element_type=jnp.float32)
    root1_ref[...] = jnp.dot(h, rw1_ref[...],
                             preferred_element_type=jnp.float32)


def _make_mid(n_pad):
    return pl.pallas_call(
        _mid_body,
        grid=(n_pad // BM,),
        in_specs=[
            pl.BlockSpec((NC, BM, F), lambda i: (0, i, 0)),
            pl.BlockSpec((BM, F), lambda i: (i, 0)),
            pl.BlockSpec((1, F), lambda i: (0, 0)),
            pl.BlockSpec((F, F), lambda i: (0, 0)),
            pl.BlockSpec((F, F), lambda i: (0, 0)),
        ],
        out_specs=[
            pl.BlockSpec((BM, F), lambda i: (i, 0)),
            pl.BlockSpec((BM, F), lambda i: (i, 0)),
        ],
        out_shape=[
            jax.ShapeDtypeStruct((n_pad, F), jnp.float32),
            jax.ShapeDtypeStruct((n_pad, F), jnp.float32),
        ],
    )


def _final_body(aggp_ref, root1_ref, b1r_ref, batch_ref, w1_ref, b1m_ref,
                w2_ref, b2_ref, out_ref, sums_sc, cnts_sc):
    i = pl.program_id(0)
    nx = jnp.maximum(aggp_ref[0] + aggp_ref[1] + root1_ref[...]
                     + b1r_ref[...], 0.0)
    bb = batch_ref[...]
    oh = (lax.broadcasted_iota(jnp.int32, (G, BM), 0) == bb
          ).astype(jnp.float32)
    psum = jnp.dot(oh, nx, preferred_element_type=jnp.float32)
    pcnt = jnp.sum(oh, axis=1, keepdims=True)

    @pl.when(i == 0)
    def _():
        sums_sc[...] = jnp.zeros_like(sums_sc)
        cnts_sc[...] = jnp.zeros_like(cnts_sc)

    sums_sc[...] += psum
    cnts_sc[...] += jnp.broadcast_to(pcnt, cnts_sc.shape)

    @pl.when(i == pl.num_programs(0) - 1)
    def _():
        gx = sums_sc[...] / jnp.maximum(cnts_sc[...], 1.0)
        hm = jnp.maximum(jnp.dot(gx, w1_ref[...],
                                 preferred_element_type=jnp.float32)
                         + b1m_ref[...], 0.0)
        out_ref[...] = jnp.dot(hm, w2_ref[...],
                               preferred_element_type=jnp.float32) + b2_ref[...]


def _make_final(n_pad, n_hid2, n_cls):
    return pl.pallas_call(
        _final_body,
        grid=(n_pad // BM,),
        in_specs=[
            pl.BlockSpec((NC, BM, F), lambda i: (0, i, 0)),
            pl.BlockSpec((BM, F), lambda i: (i, 0)),
            pl.BlockSpec((1, F), lambda i: (0, 0)),
            pl.BlockSpec((1, BM), lambda i: (0, i)),
            pl.BlockSpec((F, n_hid2), lambda i: (0, 0)),
            pl.BlockSpec((1, n_hid2), lambda i: (0, 0)),
            pl.BlockSpec((n_hid2, n_cls), lambda i: (0, 0)),
            pl.BlockSpec((1, n_cls), lambda i: (0, 0)),
        ],
        out_specs=pl.BlockSpec((G, n_cls), lambda i: (0, 0)),
        out_shape=jax.ShapeDtypeStruct((G, n_cls), jnp.float32),
        scratch_shapes=[
            pltpu.VMEM((G, F), jnp.float32),
            pltpu.VMEM((G, F), jnp.float32),
        ],
    )


def kernel(x, edge_index, edge_attr, batch, init_w0, root_w0, bias0,
           init_w1, root_w1, bias1, W1, b1, W2, b2):
    n = x.shape[0]
    e = edge_index.shape[1]
    n_hid2 = W1.shape[1]
    n_cls = W2.shape[1]

    n_pad = -(-n // (NS * CH)) * (NS * CH)
    ng = -(-e // (NW * Q * CH))
    e_pad = ng * Q * CH * NW

    w = edge_attr.reshape(-1)
    row_p = jnp.pad(edge_index[0], (0, e_pad - e)).reshape(NW, ng, Q, CH)
    col_p = jnp.pad(edge_index[1], (0, e_pad - e)).reshape(NW, ng, Q, CH)
    w_p = jnp.pad(w, (0, e_pad - e)).reshape(NW, ng, Q, CH)
    x_p = jnp.pad(x, ((0, n_pad - n), (0, 0)))
    batch_p = jnp.pad(batch, (0, n_pad - n),
                      constant_values=G).reshape(1, n_pad)

    degp = _make_deg(n_pad, ng)(col_p, w_p)
    out0, root0, dinv2 = _make_prep(n_pad)(x_p, init_w0, root_w0, degp)
    agg0, wn = _make_layer(n_pad, ng, True)(
        out0, row_p, col_p, w_p, dinv2.reshape(n_pad))
    agg0 = agg0.reshape(NC, n_pad, F)
    out1, root1 = _make_mid(n_pad)(agg0, root0, bias0.reshape(1, F),
                                   init_w1, root_w1)
    agg1, = _make_layer(n_pad, ng, False)(out1, row_p, col_p, wn)
    agg1 = agg1.reshape(NC, n_pad, F)
    logits = _make_final(n_pad, n_hid2, n_cls)(
        agg1, root1, bias1.reshape(1, F), batch_p, W1, b1.reshape(1, n_hid2),
        W2, b2.reshape(1, n_cls))
    return logits
